# Initial kernel scaffold; baseline (speedup 1.0000x reference)
import jax
import jax.numpy as jnp
from jax.experimental import pallas as pl


def kernel(x, link_feats, adj_nodes, adj_links, g1_ws, g1_wn, g1_wl, W1_self, W1_neigh, b1, g2_ws, g2_wn, g2_wl, W2_self, W2_neigh, b2, Wp, bp):
    raise NotImplementedError("write your pallas kernel here")


# speedup vs baseline: 1.5599x; 1.5599x over previous
import functools

import jax
import jax.numpy as jnp
from jax import lax
from jax.experimental import pallas as pl
from jax.experimental.pallas import tpu as pltpu
from jax.experimental.pallas import tpu_sc as plsc

N_NODES = 10000
K = 16
D = 256
DE = 16
E = N_NODES * K
OUT = 256
H1 = 2 * OUT
NCLS = 40

NW = 32
NP = 10240
NB = NP // NW
CH = 8
EC = CH * K
NCH = NB // CH
ADJ_ROWS = NP * K // 128
TILE_ROWS = ADJ_ROWS // NW
DV = OUT // 16


def _sc_mesh():
    return plsc.VectorSubcoreMesh(
        core_axis_name="c", subcore_axis_name="s", num_cores=2, num_subcores=16
    )


def _make_sc_agg():
    @functools.partial(
        pl.kernel,
        out_type=jax.ShapeDtypeStruct((NP, OUT), jnp.float32),
        mesh=_sc_mesh(),
        compiler_params=pltpu.CompilerParams(needs_layout_passes=False),
        scratch_types=[
            pltpu.VMEM((TILE_ROWS, 128), jnp.int32),
            pltpu.VMEM((TILE_ROWS, 128), jnp.int32),
            pltpu.VMEM((NB,), jnp.float32),
            pltpu.VMEM((EC, OUT), jnp.float32),
            pltpu.VMEM((EC,), jnp.float32),
            pltpu.VMEM((EC,), jnp.float32),
            pltpu.VMEM((EC,), jnp.float32),
            pltpu.VMEM((CH, OUT), jnp.float32),
            pltpu.SemaphoreType.DMA,
            pltpu.SemaphoreType.DMA,
            pltpu.SemaphoreType.DMA,
        ],
    )
    def sc_agg(adjn_hbm, adjl_hbm, ss_hbm, sn_hbm, t_hbm, hn_hbm, agg_hbm,
               an_v, al_v, own_v, rows_v, nbr_v, tev_v, gat_v, agg_v,
               sem0, sem1, sem2):
        wid = lax.axis_index("s") * 2 + lax.axis_index("c")
        row0 = wid * TILE_ROWS
        node0 = wid * NB
        pltpu.sync_copy(adjn_hbm.at[pl.ds(row0, TILE_ROWS)], an_v)
        pltpu.sync_copy(adjl_hbm.at[pl.ds(row0, TILE_ROWS)], al_v)
        pltpu.sync_copy(ss_hbm.at[pl.ds(node0, NB)], own_v)

        zero16 = jnp.zeros((16,), jnp.float32)

        def chunk_body(c, carry):
            cp0 = pltpu.async_copy(hn_hbm.at[an_v.at[c]], rows_v, sem0)
            cp1 = pltpu.async_copy(sn_hbm.at[an_v.at[c]], nbr_v, sem1)
            cp2 = pltpu.async_copy(t_hbm.at[al_v.at[c]], tev_v, sem2)
            cp0.wait()
            cp1.wait()
            cp2.wait()
            for g in range(CH):
                sn = nbr_v[pl.ds(g * 16, 16)]
                tv = tev_v[pl.ds(g * 16, 16)]
                ss = plsc.load_gather(
                    own_v, [jnp.full((16,), c * CH + g, jnp.int32)])
                score = ss + sn + tv
                gat_v[pl.ds(g * 16, 16)] = (1.0 / K) / (1.0 + jnp.exp(-score))

            def node_body(g, carry2):
                def k_body(kk, acc):
                    e = g * 16 + kk
                    gb = plsc.load_gather(gat_v, [jnp.full((16,), e, jnp.int32)])
                    return [acc[d] + gb * rows_v[e, pl.ds(d * 16, 16)]
                            for d in range(DV)]

                acc = lax.fori_loop(0, K, k_body, [zero16] * DV)
                for d in range(DV):
                    agg_v[g, pl.ds(d * 16, 16)] = acc[d]
                return carry2

            lax.fori_loop(0, CH, node_body, 0)
            pltpu.sync_copy(agg_v, agg_hbm.at[pl.ds(node0 + c * CH, CH)])
            return carry

        lax.fori_loop(0, NCH, chunk_body, 0)

    return sc_agg


_sc_agg = _make_sc_agg()


def _te_body(lf_ref, gl_ref, te_ref):
    te_ref[...] = jnp.dot(lf_ref[...], gl_ref[...],
                          preferred_element_type=jnp.float32)


def _edge_scores(lf, GL):
    blk = 2000
    return pl.pallas_call(
        _te_body,
        out_shape=jax.ShapeDtypeStruct((E, 8), jnp.float32),
        grid=(E // blk,),
        in_specs=[pl.BlockSpec((blk, DE), lambda i: (i, 0)),
                  pl.BlockSpec((DE, 8), lambda i: (0, 0))],
        out_specs=pl.BlockSpec((blk, 8), lambda i: (i, 0)),
    )(lf, GL)


def _proj1_body(x_ref, ws_ref, wn_ref, g_ref, hs_ref, hn_ref, ssn_ref):
    xb = x_ref[...]
    hs_ref[...] = jnp.dot(xb, ws_ref[...], preferred_element_type=jnp.float32)
    hn_ref[...] = jnp.dot(xb, wn_ref[...], preferred_element_type=jnp.float32)
    ssn_ref[...] = jnp.dot(xb, g_ref[...], preferred_element_type=jnp.float32)


def _layer1_proj(xp, Ws, Wn, G):
    blk = 512
    return pl.pallas_call(
        _proj1_body,
        out_shape=[jax.ShapeDtypeStruct((NP, OUT), jnp.float32),
                   jax.ShapeDtypeStruct((NP, OUT), jnp.float32),
                   jax.ShapeDtypeStruct((NP, 8), jnp.float32)],
        grid=(NP // blk,),
        in_specs=[pl.BlockSpec((blk, D), lambda i: (i, 0)),
                  pl.BlockSpec((D, OUT), lambda i: (0, 0)),
                  pl.BlockSpec((D, OUT), lambda i: (0, 0)),
                  pl.BlockSpec((D, 8), lambda i: (0, 0))],
        out_specs=[pl.BlockSpec((blk, OUT), lambda i: (i, 0)),
                   pl.BlockSpec((blk, OUT), lambda i: (i, 0)),
                   pl.BlockSpec((blk, 8), lambda i: (i, 0))],
    )(xp, Ws, Wn, G)


def _proj2_body(hs_ref, agg_ref, b_ref, ws_ref, wn_ref, g_ref,
                hs2_ref, hn2_ref, ssn2_ref):
    h1a = jnp.maximum(hs_ref[...] + b_ref[0, :OUT], 0.0)
    h1b = jnp.maximum(agg_ref[...] + b_ref[0, OUT:], 0.0)
    h1 = jnp.concatenate([h1a, h1b], axis=1)
    hs2_ref[...] = jnp.dot(h1, ws_ref[...], preferred_element_type=jnp.float32)
    hn2_ref[...] = jnp.dot(h1, wn_ref[...], preferred_element_type=jnp.float32)
    ssn2_ref[...] = jnp.dot(h1, g_ref[...], preferred_element_type=jnp.float32)


def _layer2_proj(hs1, agg1, b1, Ws, Wn, G):
    blk = 512
    return pl.pallas_call(
        _proj2_body,
        out_shape=[jax.ShapeDtypeStruct((NP, OUT), jnp.float32),
                   jax.ShapeDtypeStruct((NP, OUT), jnp.float32),
                   jax.ShapeDtypeStruct((NP, 8), jnp.float32)],
        grid=(NP // blk,),
        in_specs=[pl.BlockSpec((blk, OUT), lambda i: (i, 0)),
                  pl.BlockSpec((blk, OUT), lambda i: (i, 0)),
                  pl.BlockSpec((1, H1), lambda i: (0, 0)),
                  pl.BlockSpec((H1, OUT), lambda i: (0, 0)),
                  pl.BlockSpec((H1, OUT), lambda i: (0, 0)),
                  pl.BlockSpec((H1, 8), lambda i: (0, 0))],
        out_specs=[pl.BlockSpec((blk, OUT), lambda i: (i, 0)),
                   pl.BlockSpec((blk, OUT), lambda i: (i, 0)),
                   pl.BlockSpec((blk, 8), lambda i: (i, 0))],
    )(hs1, agg1, b1, Ws, Wn, G)


def _final_body(hs_ref, agg_ref, b_ref, wp_ref, bp_ref, out_ref):
    h2a = jnp.maximum(hs_ref[...] + b_ref[0, :OUT], 0.0)
    h2b = jnp.maximum(agg_ref[...] + b_ref[0, OUT:], 0.0)
    h2 = jnp.concatenate([h2a, h2b], axis=1)
    nrm = jnp.sqrt(jnp.sum(h2 * h2, axis=1, keepdims=True))
    hid = h2 / jnp.maximum(nrm, 1e-12)
    out_ref[...] = (jnp.dot(hid, wp_ref[...], preferred_element_type=jnp.float32)
                    + bp_ref[0, :])


def _final_proj(hs2, agg2, b2, Wp, bp):
    blk = 512
    return pl.pallas_call(
        _final_body,
        out_shape=jax.ShapeDtypeStruct((NP, NCLS), jnp.float32),
        grid=(NP // blk,),
        in_specs=[pl.BlockSpec((blk, OUT), lambda i: (i, 0)),
                  pl.BlockSpec((blk, OUT), lambda i: (i, 0)),
                  pl.BlockSpec((1, H1), lambda i: (0, 0)),
                  pl.BlockSpec((H1, NCLS), lambda i: (0, 0)),
                  pl.BlockSpec((1, NCLS), lambda i: (0, 0))],
        out_specs=pl.BlockSpec((blk, NCLS), lambda i: (i, 0)),
    )(hs2, agg2, b2, Wp, bp)


def kernel(x, link_feats, adj_nodes, adj_links,
           g1_ws, g1_wn, g1_wl, W1_self, W1_neigh, b1,
           g2_ws, g2_wn, g2_wl, W2_self, W2_neigh, b2,
           Wp, bp):
    f32 = jnp.float32
    GL = jnp.zeros((DE, 8), f32).at[:, 0].set(g1_wl).at[:, 1].set(g2_wl)
    G1 = jnp.zeros((D, 8), f32).at[:, 0].set(g1_ws).at[:, 1].set(g1_wn)
    G2 = jnp.zeros((H1, 8), f32).at[:, 0].set(g2_ws).at[:, 1].set(g2_wn)

    pad = NP - N_NODES
    xp = jnp.pad(x, ((0, pad), (0, 0)))
    adjn = jnp.pad(adj_nodes, ((0, pad), (0, 0))).reshape(ADJ_ROWS, 128)
    adjl = jnp.pad(adj_links, ((0, pad), (0, 0))).reshape(ADJ_ROWS, 128)

    te = _edge_scores(link_feats, GL)
    t1 = te[:, 0]
    t2 = te[:, 1]
    hs1, hn1, ssn1 = _layer1_proj(xp, W1_self, W1_neigh, G1)
    agg1 = _sc_agg(adjn, adjl, ssn1[:, 0], ssn1[:, 1], t1, hn1)
    hs2, hn2, ssn2 = _layer2_proj(hs1, agg1, b1[None, :], W2_self, W2_neigh, G2)
    agg2 = _sc_agg(adjn, adjl, ssn2[:, 0], ssn2[:, 1], t2, hn2)
    logits = _final_proj(hs2, agg2, b2[None, :], Wp, bp[None, :])
    return logits[:N_NODES]


# speedup vs baseline: 1.8537x; 1.1884x over previous
import functools

import jax
import jax.numpy as jnp
from jax import lax
from jax.experimental import pallas as pl
from jax.experimental.pallas import tpu as pltpu
from jax.experimental.pallas import tpu_sc as plsc

N_NODES = 10000
K = 16
D = 256
DE = 16
E = N_NODES * K
OUT = 256
H1 = 2 * OUT
NCLS = 40

NW = 32
NP = 10240
NB = NP // NW
CH = 8
EC = CH * K
NCH = NB // CH
ADJ_ROWS = NP * K // 128
TILE_ROWS = ADJ_ROWS // NW
DV = OUT // 16


def _sc_mesh():
    return plsc.VectorSubcoreMesh(
        core_axis_name="c", subcore_axis_name="s", num_cores=2, num_subcores=16
    )


def _make_sc_agg():
    @functools.partial(
        pl.kernel,
        out_type=jax.ShapeDtypeStruct((NP, OUT), jnp.float32),
        mesh=_sc_mesh(),
        compiler_params=pltpu.CompilerParams(needs_layout_passes=False),
        scratch_types=[
            pltpu.VMEM((TILE_ROWS, 128), jnp.int32),
            pltpu.VMEM((TILE_ROWS, 128), jnp.int32),
            pltpu.VMEM((NB,), jnp.float32),
            pltpu.VMEM((2, EC, OUT), jnp.float32),
            pltpu.VMEM((2, EC), jnp.float32),
            pltpu.VMEM((2, EC), jnp.float32),
            pltpu.VMEM((EC,), jnp.float32),
            pltpu.VMEM((2, CH, OUT), jnp.float32),
            pltpu.SemaphoreType.DMA,
            pltpu.SemaphoreType.DMA,
            pltpu.SemaphoreType.DMA,
            pltpu.SemaphoreType.DMA,
            pltpu.SemaphoreType.DMA,
            pltpu.SemaphoreType.DMA,
            pltpu.SemaphoreType.DMA,
            pltpu.SemaphoreType.DMA,
        ],
    )
    def sc_agg(adjn_hbm, adjl_hbm, ss_hbm, sn_hbm, t_hbm, hn_hbm, agg_hbm,
               an_v, al_v, own_v, rows_v, nbr_v, tev_v, gat_v, agg_v,
               semr0, semr1, semn0, semn1, semt0, semt1, semo0, semo1):
        wid = lax.axis_index("s") * 2 + lax.axis_index("c")
        row0 = wid * TILE_ROWS
        node0 = wid * NB
        semr = (semr0, semr1)
        semn = (semn0, semn1)
        semt = (semt0, semt1)
        semo = (semo0, semo1)
        pltpu.sync_copy(adjn_hbm.at[pl.ds(row0, TILE_ROWS)], an_v)
        pltpu.sync_copy(adjl_hbm.at[pl.ds(row0, TILE_ROWS)], al_v)
        pltpu.sync_copy(ss_hbm.at[pl.ds(node0, NB)], own_v)

        zero16 = jnp.zeros((16,), jnp.float32)

        def start_gathers(c, b):
            pltpu.async_copy(hn_hbm.at[an_v.at[c]], rows_v.at[b], semr[b])
            pltpu.async_copy(sn_hbm.at[an_v.at[c]], nbr_v.at[b], semn[b])
            pltpu.async_copy(t_hbm.at[al_v.at[c]], tev_v.at[b], semt[b])

        def wait_gathers(c, b):
            pltpu.make_async_copy(hn_hbm.at[an_v.at[c]], rows_v.at[b],
                                  semr[b]).wait()
            pltpu.make_async_copy(sn_hbm.at[an_v.at[c]], nbr_v.at[b],
                                  semn[b]).wait()
            pltpu.make_async_copy(t_hbm.at[al_v.at[c]], tev_v.at[b],
                                  semt[b]).wait()

        def compute_chunk(c, b):
            for g in range(CH):
                sn = nbr_v[b, pl.ds(g * 16, 16)]
                tv = tev_v[b, pl.ds(g * 16, 16)]
                ss = plsc.load_gather(
                    own_v, [jnp.full((16,), c * CH + g, jnp.int32)])
                score = ss + sn + tv
                gat_v[pl.ds(g * 16, 16)] = (1.0 / K) / (1.0 + jnp.exp(-score))

            def node_body(g, carry2):
                acc = [zero16] * DV
                for kk in range(K):
                    e = g * 16 + kk
                    gb = plsc.load_gather(gat_v, [jnp.full((16,), e, jnp.int32)])
                    acc = [acc[d] + gb * rows_v[b, e, pl.ds(d * 16, 16)]
                           for d in range(DV)]
                for d in range(DV):
                    agg_v[b, g, pl.ds(d * 16, 16)] = acc[d]
                return carry2

            lax.fori_loop(0, CH, node_body, 0, unroll=False)

        start_gathers(0, 0)

        def chunk_pair(c2, carry):
            for b in range(2):
                c = 2 * c2 + b
                nxt = c + 1

                @pl.when(nxt < NCH)
                def _():
                    start_gathers(nxt, 1 - b)

                wait_gathers(c, b)

                @pl.when(c >= 2)
                def _():
                    pltpu.make_async_copy(
                        agg_v.at[b],
                        agg_hbm.at[pl.ds(node0 + c * CH, CH)],
                        semo[b]).wait()

                compute_chunk(c, b)
                pltpu.async_copy(agg_v.at[b],
                                 agg_hbm.at[pl.ds(node0 + c * CH, CH)],
                                 semo[b])
            return carry

        lax.fori_loop(0, NCH // 2, chunk_pair, 0, unroll=False)
        for b in range(2):
            pltpu.make_async_copy(agg_v.at[b],
                                  agg_hbm.at[pl.ds(node0, CH)],
                                  semo[b]).wait()

    return sc_agg


_sc_agg = _make_sc_agg()


def _te_body(lf_ref, gl_ref, te_ref):
    te_ref[...] = jnp.dot(lf_ref[...], gl_ref[...],
                          preferred_element_type=jnp.float32)


def _edge_scores(lf, GL):
    blk = 2000
    return pl.pallas_call(
        _te_body,
        out_shape=jax.ShapeDtypeStruct((E, 8), jnp.float32),
        grid=(E // blk,),
        in_specs=[pl.BlockSpec((blk, DE), lambda i: (i, 0)),
                  pl.BlockSpec((DE, 8), lambda i: (0, 0))],
        out_specs=pl.BlockSpec((blk, 8), lambda i: (i, 0)),
    )(lf, GL)


def _proj1_body(x_ref, ws_ref, wn_ref, g_ref, hs_ref, hn_ref, ssn_ref):
    xb = x_ref[...]
    hs_ref[...] = jnp.dot(xb, ws_ref[...], preferred_element_type=jnp.float32)
    hn_ref[...] = jnp.dot(xb, wn_ref[...], preferred_element_type=jnp.float32)
    ssn_ref[...] = jnp.dot(xb, g_ref[...], preferred_element_type=jnp.float32)


def _layer1_proj(xp, Ws, Wn, G):
    blk = 512
    return pl.pallas_call(
        _proj1_body,
        out_shape=[jax.ShapeDtypeStruct((NP, OUT), jnp.float32),
                   jax.ShapeDtypeStruct((NP, OUT), jnp.float32),
                   jax.ShapeDtypeStruct((NP, 8), jnp.float32)],
        grid=(NP // blk,),
        in_specs=[pl.BlockSpec((blk, D), lambda i: (i, 0)),
                  pl.BlockSpec((D, OUT), lambda i: (0, 0)),
                  pl.BlockSpec((D, OUT), lambda i: (0, 0)),
                  pl.BlockSpec((D, 8), lambda i: (0, 0))],
        out_specs=[pl.BlockSpec((blk, OUT), lambda i: (i, 0)),
                   pl.BlockSpec((blk, OUT), lambda i: (i, 0)),
                   pl.BlockSpec((blk, 8), lambda i: (i, 0))],
    )(xp, Ws, Wn, G)


def _proj2_body(hs_ref, agg_ref, b_ref, ws_ref, wn_ref, g_ref,
                hs2_ref, hn2_ref, ssn2_ref):
    h1a = jnp.maximum(hs_ref[...] + b_ref[0, :OUT], 0.0)
    h1b = jnp.maximum(agg_ref[...] + b_ref[0, OUT:], 0.0)
    h1 = jnp.concatenate([h1a, h1b], axis=1)
    hs2_ref[...] = jnp.dot(h1, ws_ref[...], preferred_element_type=jnp.float32)
    hn2_ref[...] = jnp.dot(h1, wn_ref[...], preferred_element_type=jnp.float32)
    ssn2_ref[...] = jnp.dot(h1, g_ref[...], preferred_element_type=jnp.float32)


def _layer2_proj(hs1, agg1, b1, Ws, Wn, G):
    blk = 512
    return pl.pallas_call(
        _proj2_body,
        out_shape=[jax.ShapeDtypeStruct((NP, OUT), jnp.float32),
                   jax.ShapeDtypeStruct((NP, OUT), jnp.float32),
                   jax.ShapeDtypeStruct((NP, 8), jnp.float32)],
        grid=(NP // blk,),
        in_specs=[pl.BlockSpec((blk, OUT), lambda i: (i, 0)),
                  pl.BlockSpec((blk, OUT), lambda i: (i, 0)),
                  pl.BlockSpec((1, H1), lambda i: (0, 0)),
                  pl.BlockSpec((H1, OUT), lambda i: (0, 0)),
                  pl.BlockSpec((H1, OUT), lambda i: (0, 0)),
                  pl.BlockSpec((H1, 8), lambda i: (0, 0))],
        out_specs=[pl.BlockSpec((blk, OUT), lambda i: (i, 0)),
                   pl.BlockSpec((blk, OUT), lambda i: (i, 0)),
                   pl.BlockSpec((blk, 8), lambda i: (i, 0))],
    )(hs1, agg1, b1, Ws, Wn, G)


def _final_body(hs_ref, agg_ref, b_ref, wp_ref, bp_ref, out_ref):
    h2a = jnp.maximum(hs_ref[...] + b_ref[0, :OUT], 0.0)
    h2b = jnp.maximum(agg_ref[...] + b_ref[0, OUT:], 0.0)
    h2 = jnp.concatenate([h2a, h2b], axis=1)
    nrm = jnp.sqrt(jnp.sum(h2 * h2, axis=1, keepdims=True))
    hid = h2 / jnp.maximum(nrm, 1e-12)
    out_ref[...] = (jnp.dot(hid, wp_ref[...], preferred_element_type=jnp.float32)
                    + bp_ref[0, :])


def _final_proj(hs2, agg2, b2, Wp, bp):
    blk = 512
    return pl.pallas_call(
        _final_body,
        out_shape=jax.ShapeDtypeStruct((NP, NCLS), jnp.float32),
        grid=(NP // blk,),
        in_specs=[pl.BlockSpec((blk, OUT), lambda i: (i, 0)),
                  pl.BlockSpec((blk, OUT), lambda i: (i, 0)),
                  pl.BlockSpec((1, H1), lambda i: (0, 0)),
                  pl.BlockSpec((H1, NCLS), lambda i: (0, 0)),
                  pl.BlockSpec((1, NCLS), lambda i: (0, 0))],
        out_specs=pl.BlockSpec((blk, NCLS), lambda i: (i, 0)),
    )(hs2, agg2, b2, Wp, bp)


def kernel(x, link_feats, adj_nodes, adj_links,
           g1_ws, g1_wn, g1_wl, W1_self, W1_neigh, b1,
           g2_ws, g2_wn, g2_wl, W2_self, W2_neigh, b2,
           Wp, bp):
    f32 = jnp.float32
    GL = jnp.zeros((DE, 8), f32).at[:, 0].set(g1_wl).at[:, 1].set(g2_wl)
    G1 = jnp.zeros((D, 8), f32).at[:, 0].set(g1_ws).at[:, 1].set(g1_wn)
    G2 = jnp.zeros((H1, 8), f32).at[:, 0].set(g2_ws).at[:, 1].set(g2_wn)

    pad = NP - N_NODES
    xp = jnp.pad(x, ((0, pad), (0, 0)))
    adjn = jnp.pad(adj_nodes, ((0, pad), (0, 0))).reshape(ADJ_ROWS, 128)
    adjl = jnp.pad(adj_links, ((0, pad), (0, 0))).reshape(ADJ_ROWS, 128)

    te = _edge_scores(link_feats, GL)
    t1 = te[:, 0]
    t2 = te[:, 1]
    hs1, hn1, ssn1 = _layer1_proj(xp, W1_self, W1_neigh, G1)
    agg1 = _sc_agg(adjn, adjl, ssn1[:, 0], ssn1[:, 1], t1, hn1)
    hs2, hn2, ssn2 = _layer2_proj(hs1, agg1, b1[None, :], W2_self, W2_neigh, G2)
    agg2 = _sc_agg(adjn, adjl, ssn2[:, 0], ssn2[:, 1], t2, hn2)
    logits = _final_proj(hs2, agg2, b2[None, :], Wp, bp[None, :])
    return logits[:N_NODES]
